# baseline (speedup 1.0000x reference)
import functools

import jax
import jax.numpy as jnp
from jax import lax
from jax.experimental import pallas as pl
from jax.experimental.pallas import tpu as pltpu
from jax.experimental.pallas import tpu_sc as plsc

DIM = 32
CHUNK = 128


def _body(n_chunks_per_worker, num_cores, idx_hbm, weight_hbm, out_hbm,
          idx_v, rows_v, sem):
    wid = lax.axis_index("s") * num_cores + lax.axis_index("c")
    pltpu.sync_copy(idx_hbm.at[wid], idx_v)
    copies = [
        pltpu.async_copy(weight_hbm.at[idx_v.at[j]], rows_v.at[j], sem)
        for j in range(n_chunks_per_worker)
    ]
    for c in copies:
        c.wait()
    pltpu.sync_copy(rows_v, out_hbm.at[wid])


def kernel(indices, weight):
    batch, n_fields = indices.shape
    total = batch * n_fields
    assert total % CHUNK == 0
    n_chunks = total // CHUNK

    info = plsc.get_sparse_core_info()
    num_workers = info.num_cores * info.num_subcores
    assert n_chunks % num_workers == 0
    n_per_w = n_chunks // num_workers

    idx_flat = indices.reshape(num_workers, n_per_w, CHUNK).astype(jnp.int32)

    mesh = plsc.VectorSubcoreMesh(core_axis_name="c", subcore_axis_name="s")
    k = pl.kernel(
        functools.partial(_body, n_per_w, info.num_cores),
        out_type=jax.ShapeDtypeStruct((num_workers, n_per_w, CHUNK, DIM),
                                      jnp.float32),
        mesh=mesh,
        compiler_params=pltpu.CompilerParams(use_tc_tiling_on_sc=False),
        scratch_types=[
            pltpu.VMEM((n_per_w, CHUNK), jnp.int32),
            pltpu.VMEM((n_per_w, CHUNK, DIM), jnp.float32),
            pltpu.SemaphoreType.DMA,
        ],
    )
    out = k(idx_flat, weight)
    return out.reshape(batch, n_fields, DIM)
